# baseline (speedup 1.0000x reference)
import functools

import jax
import jax.numpy as jnp
from jax import lax
from jax.experimental import pallas as pl
from jax.experimental.pallas import tpu as pltpu
from jax.experimental.pallas import tpu_sc as plsc

EDGE_VOCAB = 64
EDGE_DIM = 128
HIDDEN_DIM = 256
B, N = 16, 64
B_TOT = B * N * N
N_REPLICAS = 32

SC_ROWS = 4096
TC_BLK = 2048
CP_BLK = 2048


def _mlp_table_kernel(table_ref, w1_ref, b1_ref, w2_ref, b2_ref, out_ref):
    h = jnp.dot(table_ref[...], w1_ref[...], preferred_element_type=jnp.float32)
    h = h + b1_ref[...]
    h = jax.nn.gelu(h)
    o = jnp.dot(h, w2_ref[...], preferred_element_type=jnp.float32)
    o = o + b2_ref[...]
    out_ref[...] = jnp.broadcast_to(
        o[None], (N_REPLICAS, EDGE_VOCAB, HIDDEN_DIM)
    ).reshape(N_REPLICAS * EDGE_VOCAB, HIDDEN_DIM)


def _fused_table(table, W1, b1, W2, b2):
    return pl.pallas_call(
        _mlp_table_kernel,
        out_shape=jax.ShapeDtypeStruct(
            (N_REPLICAS * EDGE_VOCAB, HIDDEN_DIM), jnp.float32
        ),
    )(table, W1, b1.reshape(1, HIDDEN_DIM), W2, b2.reshape(1, HIDDEN_DIM))


N_BUF = 4
CHUNK = 64
GATHER_LAG = 2


def _make_gather(n_rows):
    info = plsc.get_sparse_core_info()
    NC, NS = info.num_cores, info.num_subcores
    NW = NC * NS
    b_per_w = n_rows // NW
    n_chunks = b_per_w // CHUNK
    mesh = plsc.VectorSubcoreMesh(core_axis_name="c", subcore_axis_name="s")

    @functools.partial(
        pl.kernel,
        mesh=mesh,
        out_type=jax.ShapeDtypeStruct((n_rows, HIDDEN_DIM), jnp.float32),
        scratch_types=[
            pltpu.VMEM((b_per_w,), jnp.int32),
        ]
        + [pltpu.VMEM((CHUNK, HIDDEN_DIM), jnp.float32) for _ in range(N_BUF)]
        + [pltpu.SemaphoreType.DMA for _ in range(2 * N_BUF)],
    )
    def gather_k(idx_hbm, table_hbm, out_hbm, idx_v, *bufs_and_sems):
        bufs = bufs_and_sems[:N_BUF]
        g_sems = bufs_and_sems[N_BUF : 2 * N_BUF]
        w_sems = bufs_and_sems[2 * N_BUF :]
        wid = lax.axis_index("s") * NC + lax.axis_index("c")
        base = wid * b_per_w
        pltpu.sync_copy(idx_hbm.at[pl.ds(base, b_per_w)], idx_v)
        off = (wid * EDGE_VOCAB).astype(jnp.int32)
        for j in range(b_per_w // 16):
            sl = pl.ds(j * 16, 16)
            idx_v[sl] = idx_v[sl] + off

        def start_gather(i):
            return pltpu.async_copy(
                table_hbm.at[idx_v.at[pl.ds(i * CHUNK, CHUNK)]],
                bufs[i % N_BUF],
                g_sems[i % N_BUF],
            )

        def start_write(i):
            return pltpu.async_copy(
                bufs[i % N_BUF],
                out_hbm.at[pl.ds(base + i * CHUNK, CHUNK)],
                w_sems[i % N_BUF],
            )

        g_handles, w_handles = {}, {}
        for s in range(n_chunks + GATHER_LAG):
            i_g = s
            if i_g < n_chunks:
                if i_g >= N_BUF:
                    w_handles.pop(i_g - N_BUF).wait()
                g_handles[i_g] = start_gather(i_g)
            i_w = s - GATHER_LAG
            if 0 <= i_w < n_chunks:
                g_handles.pop(i_w).wait()
                w_handles[i_w] = start_write(i_w)
        for i in sorted(w_handles):
            w_handles.pop(i).wait()

    return gather_k


def _onehot_kernel(idx_ref, tab_ref, out_ref):
    sub = TC_BLK // 128
    onehot = (
        idx_ref[...][..., None]
        == lax.broadcasted_iota(jnp.int32, (sub, 128, EDGE_VOCAB), 2)
    ).astype(jnp.float32).reshape(TC_BLK, EDGE_VOCAB)
    out_ref[...] = jnp.dot(onehot, tab_ref[...], preferred_element_type=jnp.float32)


def _tc_lookup(idx2d, fused):
    n_blocks = (B_TOT - SC_ROWS) // TC_BLK
    blk0 = SC_ROWS // TC_BLK
    return pl.pallas_call(
        _onehot_kernel,
        grid=(n_blocks,),
        in_specs=[
            pl.BlockSpec((TC_BLK // 128, 128), lambda j: (blk0 + j, 0)),
            pl.BlockSpec((EDGE_VOCAB, HIDDEN_DIM), lambda j: (0, 0)),
        ],
        out_specs=pl.BlockSpec((TC_BLK, HIDDEN_DIM), lambda j: (blk0 + j, 0)),
        out_shape=jax.ShapeDtypeStruct((B_TOT, HIDDEN_DIM), jnp.float32),
    )(idx2d, fused)


def _stitch_kernel(full_ref, sc_ref, out_ref):
    del full_ref
    out_ref[...] = sc_ref[...]


def _stitch(tc_full, sc_out):
    return pl.pallas_call(
        _stitch_kernel,
        grid=(SC_ROWS // CP_BLK,),
        in_specs=[
            pl.BlockSpec(memory_space=pl.ANY),
            pl.BlockSpec((CP_BLK, HIDDEN_DIM), lambda j: (j, 0)),
        ],
        out_specs=pl.BlockSpec((CP_BLK, HIDDEN_DIM), lambda j: (j, 0)),
        out_shape=jax.ShapeDtypeStruct((B_TOT, HIDDEN_DIM), jnp.float32),
        input_output_aliases={0: 0},
    )(tc_full, sc_out)


def kernel(edge_types, table, W1, b1, W2, b2):
    fused = _fused_table(table, W1, b1, W2, b2)
    idx = edge_types.reshape(B_TOT).astype(jnp.int32)
    sc_out = _make_gather(SC_ROWS)(idx, fused)
    tc_full = _tc_lookup(idx.reshape(B_TOT // 128, 128), fused)
    out = lax.dynamic_update_slice(tc_full, sc_out, (0, 0))
    return out.reshape(B, N, N, HIDDEN_DIM)


# speedup vs baseline: 1.1826x; 1.1826x over previous
import functools

import jax
import jax.numpy as jnp
from jax import lax
from jax.experimental import pallas as pl
from jax.experimental.pallas import tpu as pltpu
from jax.experimental.pallas import tpu_sc as plsc

EDGE_VOCAB = 64
EDGE_DIM = 128
HIDDEN_DIM = 256
B, N = 16, 64
B_TOT = B * N * N
N_REPLICAS = 32

SC_ROWS = 4096
TC_BLK = 4096
CP_BLK = 2048


def _mlp_table_kernel(table_ref, w1_ref, b1_ref, w2_ref, b2_ref, out_ref):
    h = jnp.dot(table_ref[...], w1_ref[...], preferred_element_type=jnp.float32)
    h = h + b1_ref[...]
    h = jax.nn.gelu(h)
    o = jnp.dot(h, w2_ref[...], preferred_element_type=jnp.float32)
    o = o + b2_ref[...]
    out_ref[...] = jnp.broadcast_to(
        o[None], (N_REPLICAS, EDGE_VOCAB, HIDDEN_DIM)
    ).reshape(N_REPLICAS * EDGE_VOCAB, HIDDEN_DIM)


def _fused_table(table, W1, b1, W2, b2):
    return pl.pallas_call(
        _mlp_table_kernel,
        out_shape=jax.ShapeDtypeStruct(
            (N_REPLICAS * EDGE_VOCAB, HIDDEN_DIM), jnp.float32
        ),
    )(table, W1, b1.reshape(1, HIDDEN_DIM), W2, b2.reshape(1, HIDDEN_DIM))


N_BUF = 4
CHUNK = 64
GATHER_LAG = 2


def _make_gather(n_rows):
    info = plsc.get_sparse_core_info()
    NC, NS = info.num_cores, info.num_subcores
    NW = NC * NS
    b_per_w = n_rows // NW
    assert b_per_w == 128
    n_chunks = b_per_w // CHUNK
    mesh = plsc.VectorSubcoreMesh(core_axis_name="c", subcore_axis_name="s")

    @functools.partial(
        pl.kernel,
        mesh=mesh,
        out_type=jax.ShapeDtypeStruct((n_rows, HIDDEN_DIM), jnp.float32),
        scratch_types=[
            pltpu.VMEM((b_per_w,), jnp.int32),
        ]
        + [pltpu.VMEM((CHUNK, HIDDEN_DIM), jnp.float32) for _ in range(N_BUF)]
        + [pltpu.SemaphoreType.DMA for _ in range(2 * N_BUF)],
    )
    def gather_k(idx_hbm, table_hbm, out_hbm, idx_v, *bufs_and_sems):
        bufs = bufs_and_sems[:N_BUF]
        g_sems = bufs_and_sems[N_BUF : 2 * N_BUF]
        w_sems = bufs_and_sems[2 * N_BUF :]
        wid = lax.axis_index("s") * NC + lax.axis_index("c")
        base = wid * b_per_w
        pltpu.sync_copy(idx_hbm.at[wid], idx_v)
        off = (wid * EDGE_VOCAB).astype(jnp.int32)
        for j in range(b_per_w // 16):
            sl = pl.ds(j * 16, 16)
            idx_v[sl] = idx_v[sl] + off

        def start_gather(i):
            return pltpu.async_copy(
                table_hbm.at[idx_v.at[pl.ds(i * CHUNK, CHUNK)]],
                bufs[i % N_BUF],
                g_sems[i % N_BUF],
            )

        def start_write(i):
            return pltpu.async_copy(
                bufs[i % N_BUF],
                out_hbm.at[pl.ds(base + i * CHUNK, CHUNK)],
                w_sems[i % N_BUF],
            )

        g_handles, w_handles = {}, {}
        for s in range(n_chunks + GATHER_LAG):
            i_g = s
            if i_g < n_chunks:
                if i_g >= N_BUF:
                    w_handles.pop(i_g - N_BUF).wait()
                g_handles[i_g] = start_gather(i_g)
            i_w = s - GATHER_LAG
            if 0 <= i_w < n_chunks:
                g_handles.pop(i_w).wait()
                w_handles[i_w] = start_write(i_w)
        for i in sorted(w_handles):
            w_handles.pop(i).wait()

    return gather_k


def _onehot_kernel(
    idx_ref, table_ref, w1_ref, b1_ref, w2_ref, b2_ref, out_ref, tab_s
):
    @pl.when(pl.program_id(0) == 0)
    def _():
        h = jnp.dot(
            table_ref[...], w1_ref[...], preferred_element_type=jnp.float32
        )
        h = jax.nn.gelu(h + b1_ref[...])
        o = jnp.dot(h, w2_ref[...], preferred_element_type=jnp.float32)
        tab_s[...] = o + b2_ref[...]

    sub = TC_BLK // 128
    onehot = (
        idx_ref[...][..., None]
        == lax.broadcasted_iota(jnp.int32, (sub, 128, EDGE_VOCAB), 2)
    ).astype(jnp.float32).reshape(TC_BLK, EDGE_VOCAB)
    out_ref[...] = jnp.dot(onehot, tab_s[...], preferred_element_type=jnp.float32)


def _tc_lookup(idx2d, table, W1, b1, W2, b2):
    n_blocks = (B_TOT - SC_ROWS) // TC_BLK
    blk0 = SC_ROWS // TC_BLK
    fixed = lambda j: (0, 0)
    return pl.pallas_call(
        _onehot_kernel,
        grid=(n_blocks,),
        in_specs=[
            pl.BlockSpec((TC_BLK // 128, 128), lambda j: (blk0 + j, 0)),
            pl.BlockSpec((EDGE_VOCAB, EDGE_DIM), fixed),
            pl.BlockSpec((EDGE_DIM, HIDDEN_DIM), fixed),
            pl.BlockSpec((1, HIDDEN_DIM), fixed),
            pl.BlockSpec((HIDDEN_DIM, HIDDEN_DIM), fixed),
            pl.BlockSpec((1, HIDDEN_DIM), fixed),
        ],
        out_specs=pl.BlockSpec((TC_BLK, HIDDEN_DIM), lambda j: (blk0 + j, 0)),
        out_shape=jax.ShapeDtypeStruct((B_TOT, HIDDEN_DIM), jnp.float32),
        scratch_shapes=[pltpu.VMEM((EDGE_VOCAB, HIDDEN_DIM), jnp.float32)],
    )(
        idx2d,
        table,
        W1,
        b1.reshape(1, HIDDEN_DIM),
        W2,
        b2.reshape(1, HIDDEN_DIM),
    )


def _stitch_kernel(full_ref, sc_ref, out_ref):
    del full_ref
    out_ref[...] = sc_ref[...]


def _stitch(tc_full, sc_out):
    return pl.pallas_call(
        _stitch_kernel,
        grid=(SC_ROWS // CP_BLK,),
        in_specs=[
            pl.BlockSpec(memory_space=pl.ANY),
            pl.BlockSpec((CP_BLK, HIDDEN_DIM), lambda j: (j, 0)),
        ],
        out_specs=pl.BlockSpec((CP_BLK, HIDDEN_DIM), lambda j: (j, 0)),
        out_shape=jax.ShapeDtypeStruct((B_TOT, HIDDEN_DIM), jnp.float32),
        input_output_aliases={0: 0},
    )(tc_full, sc_out)


def kernel(edge_types, table, W1, b1, W2, b2):
    fused = _fused_table(table, W1, b1, W2, b2)
    idx2d = edge_types.reshape(B_TOT // 128, 128).astype(jnp.int32)
    sc_out = _make_gather(SC_ROWS)(idx2d, fused)
    tc_full = _tc_lookup(idx2d, table, W1, b1, W2, b2)
    out = lax.dynamic_update_slice(tc_full, sc_out, (0, 0))
    return out.reshape(B, N, N, HIDDEN_DIM)
